# Initial kernel scaffold; baseline (speedup 1.0000x reference)
import functools

import jax
import jax.numpy as jnp
from jax import lax
from jax.experimental import pallas as pl
from jax.experimental.pallas import tpu as pltpu
from jax.experimental.pallas import tpu_sc as plsc

N = 10000
E = 320000
F = 64
H = 128
NC = 2
NS = 16

E_PER_CORE = E // NC
E_PER_TILE = E_PER_CORE // NS
DEG_CHUNK = 2000
MSG_CHUNK = 1000
ROWS_PER_TILE = N // NS


def _sc_degrees(src0, dst0, src1, dst1, zdeg):
    mesh = plsc.VectorSubcoreMesh(core_axis_name="c", subcore_axis_name="s")

    @functools.partial(
        pl.kernel,
        out_type=jax.ShapeDtypeStruct((NC, 4, N), jnp.float32),
        mesh=mesh,
        scratch_types=[
            pltpu.VMEM_SHARED((N,), jnp.float32),
            pltpu.VMEM_SHARED((N,), jnp.float32),
            pltpu.VMEM_SHARED((N,), jnp.float32),
            pltpu.VMEM_SHARED((N,), jnp.float32),
            pltpu.VMEM((DEG_CHUNK,), jnp.int32),
            pltpu.VMEM((DEG_CHUNK,), jnp.float32),
        ],
    )
    def k(src0_h, dst0_h, src1_h, dst1_h, zdeg_h, out_h,
          d00, d10, d01, d11, idx_v, ones_v):
        cid = lax.axis_index("c")
        sid = lax.axis_index("s")

        @pl.when(sid == 0)
        def _():
            pltpu.sync_copy(zdeg_h, d00)
            pltpu.sync_copy(zdeg_h, d10)
            pltpu.sync_copy(zdeg_h, d01)
            pltpu.sync_copy(zdeg_h, d11)

        def fill(i, _):
            ones_v[pl.ds(pl.multiple_of(i * 16, 16), 16)] = jnp.full(
                (16,), 1.0, jnp.float32)
            return 0
        lax.fori_loop(0, DEG_CHUNK // 16, fill, 0)
        plsc.subcore_barrier()

        base0 = cid * E_PER_CORE + sid * E_PER_TILE
        for eh, dscr in ((src0_h, d00), (dst0_h, d10),
                         (src1_h, d01), (dst1_h, d11)):
            def body(kk, _, eh=eh, dscr=dscr):
                b = pl.multiple_of(base0 + kk * DEG_CHUNK, 8)
                pltpu.sync_copy(eh.at[pl.ds(b, DEG_CHUNK)], idx_v)
                pltpu.sync_copy(ones_v, dscr.at[idx_v], add=True)
                return 0
            lax.fori_loop(0, E_PER_TILE // DEG_CHUNK, body, 0)
        plsc.subcore_barrier()

        @pl.when(sid < 10)
        def _():
            o = pl.multiple_of(sid * 1000, 8)
            for j, dscr in enumerate((d00, d10, d01, d11)):
                pltpu.sync_copy(dscr.at[pl.ds(o, 1000)],
                                out_h.at[cid, j, pl.ds(o, 1000)])

    return k(src0, dst0, src1, dst1, zdeg)


def _sc_messages(h0, h1, src0, dst0, src1, dst1, zagg):
    mesh = plsc.VectorSubcoreMesh(core_axis_name="c", subcore_axis_name="s")

    @functools.partial(
        pl.kernel,
        out_type=jax.ShapeDtypeStruct((NC, 2, N, F), jnp.float32),
        mesh=mesh,
        scratch_types=[
            pltpu.VMEM_SHARED((N, F), jnp.float32),
            pltpu.VMEM_SHARED((N, F), jnp.float32),
            pltpu.VMEM((MSG_CHUNK,), jnp.int32),
            pltpu.VMEM((MSG_CHUNK,), jnp.int32),
            pltpu.VMEM((MSG_CHUNK, F), jnp.float32),
            pltpu.SemaphoreType.DMA,
        ],
    )
    def k(h0_h, h1_h, src0_h, dst0_h, src1_h, dst1_h, zagg_h, out_h,
          agg0, agg1, idxs_v, idxd_v, rows_v, sem):
        cid = lax.axis_index("c")
        sid = lax.axis_index("s")

        @pl.when(sid == 0)
        def _():
            pltpu.sync_copy(zagg_h, agg0)
            pltpu.sync_copy(zagg_h, agg1)
        plsc.subcore_barrier()

        base0 = cid * E_PER_CORE + sid * E_PER_TILE
        for sh, dh, hh, agg in ((src0_h, dst0_h, h0_h, agg0),
                                (src1_h, dst1_h, h1_h, agg1)):
            def body(kk, _, sh=sh, dh=dh, hh=hh, agg=agg):
                b = pl.multiple_of(base0 + kk * MSG_CHUNK, 8)
                pltpu.sync_copy(sh.at[pl.ds(b, MSG_CHUNK)], idxs_v)
                pltpu.sync_copy(dh.at[pl.ds(b, MSG_CHUNK)], idxd_v)
                pltpu.async_copy(hh.at[idxs_v], rows_v, sem).wait()
                pltpu.sync_copy(rows_v, agg.at[idxd_v], add=True)
                return 0
            lax.fori_loop(0, E_PER_TILE // MSG_CHUNK, body, 0)
        plsc.subcore_barrier()

        o = sid * ROWS_PER_TILE
        for g, agg in enumerate((agg0, agg1)):
            pltpu.sync_copy(agg.at[pl.ds(o, ROWS_PER_TILE)],
                            out_h.at[cid, g, pl.ds(o, ROWS_PER_TILE)])

    return k(h0, h1, src0, dst0, src1, dst1, zagg)


def _tc_h(x_st, od_st):
    def body(x_ref, od_ref, h_ref):
        deg = od_ref[0, :, 0:1] + od_ref[0, :, 1:2]
        norm = lax.rsqrt(jnp.maximum(deg, 1.0))
        h_ref[0] = x_ref[0] * norm

    return pl.pallas_call(
        body,
        grid=(2,),
        in_specs=[
            pl.BlockSpec((1, N, F), lambda g: (g, 0, 0)),
            pl.BlockSpec((1, N, 2), lambda g: (g, 0, 0)),
        ],
        out_specs=pl.BlockSpec((1, N, F), lambda g: (g, 0, 0)),
        out_shape=jax.ShapeDtypeStruct((2, N, F), jnp.float32),
    )(x_st, od_st)


def _tc_final(agg_parts, id_st, W, b2):
    def body(ap_ref, id_ref, w_ref, b_ref, out_ref):
        agg = ap_ref[0, 0] + ap_ref[1, 0]
        deg = id_ref[0, :, 0:1] + id_ref[0, :, 1:2]
        norm = lax.rsqrt(jnp.maximum(deg, 1.0))
        out_ref[0] = jnp.dot(agg * norm, w_ref[...],
                             preferred_element_type=jnp.float32) + b_ref[...]

    return pl.pallas_call(
        body,
        grid=(2,),
        in_specs=[
            pl.BlockSpec((NC, 1, N, F), lambda g: (0, g, 0, 0)),
            pl.BlockSpec((1, N, 2), lambda g: (g, 0, 0)),
            pl.BlockSpec((F, H), lambda g: (0, 0)),
            pl.BlockSpec((1, H), lambda g: (0, 0)),
        ],
        out_specs=pl.BlockSpec((1, N, H), lambda g: (g, 0, 0)),
        out_shape=jax.ShapeDtypeStruct((2, N, H), jnp.float32),
    )(agg_parts, id_st, W, b2)


def kernel(feats0, feats1, W, b, edge_index0, edge_index1):
    src0 = edge_index0[0]
    dst0 = edge_index0[1]
    src1 = edge_index1[0]
    dst1 = edge_index1[1]
    zdeg = jnp.zeros((N,), jnp.float32)
    zagg = jnp.zeros((N, F), jnp.float32)

    dp = _sc_degrees(src0, dst0, src1, dst1, zdeg)
    od_st = jnp.stack([dp[:, 0, :].T, dp[:, 2, :].T])
    id_st = jnp.stack([dp[:, 1, :].T, dp[:, 3, :].T])

    x_st = jnp.stack([feats0, feats1])
    h_st = _tc_h(x_st, od_st)

    ap = _sc_messages(h_st[0], h_st[1], src0, dst0, src1, dst1, zagg)
    out = _tc_final(ap, id_st, W, b.reshape(1, H))
    return (out[0], out[1])


# speedup vs baseline: 5.2734x; 5.2734x over previous
import functools

import jax
import jax.numpy as jnp
from jax import lax
from jax.experimental import pallas as pl
from jax.experimental.pallas import tpu as pltpu
from jax.experimental.pallas import tpu_sc as plsc

N = 10000
E = 320000
F = 64
H = 128
NC = 2
NS = 16

LANE = 128
EROWS = 2560
EPAD = EROWS * LANE
ROWS_PER_TILE = EROWS // (NC * NS)
NP = N + 8


def _sc_degrees(s0, d0, s1, d1, zdeg):
    mesh = plsc.VectorSubcoreMesh(core_axis_name="c", subcore_axis_name="s")

    @functools.partial(
        pl.kernel,
        out_type=jax.ShapeDtypeStruct((NC, 4, N), jnp.float32),
        mesh=mesh,
        scratch_types=[
            pltpu.VMEM_SHARED((NP,), jnp.float32),
            pltpu.VMEM_SHARED((NP,), jnp.float32),
            pltpu.VMEM_SHARED((NP,), jnp.float32),
            pltpu.VMEM_SHARED((NP,), jnp.float32),
            pltpu.VMEM((ROWS_PER_TILE, LANE), jnp.int32),
            pltpu.VMEM((LANE,), jnp.float32),
        ],
        compiler_params=pltpu.CompilerParams(use_tc_tiling_on_sc=False),
    )
    def k(s0_h, d0_h, s1_h, d1_h, zdeg_h, out_h,
          a00, a10, a01, a11, idx_v, ones_v):
        cid = lax.axis_index("c")
        sid = lax.axis_index("s")

        @pl.when(sid == 0)
        def _():
            pltpu.sync_copy(zdeg_h, a00)
            pltpu.sync_copy(zdeg_h, a10)
            pltpu.sync_copy(zdeg_h, a01)
            pltpu.sync_copy(zdeg_h, a11)

        for i in range(LANE // 16):
            ones_v[pl.ds(i * 16, 16)] = jnp.full((16,), 1.0, jnp.float32)
        plsc.subcore_barrier()

        rowbase = (cid * NS + sid) * ROWS_PER_TILE
        for eh, acc in ((s0_h, a00), (d0_h, a10), (s1_h, a01), (d1_h, a11)):
            pltpu.sync_copy(eh.at[pl.ds(rowbase, ROWS_PER_TILE)], idx_v)

            def body(j, _, acc=acc):
                pltpu.sync_copy(ones_v, acc.at[idx_v.at[j]], add=True)
                return 0
            lax.fori_loop(0, ROWS_PER_TILE, body, 0)
        plsc.subcore_barrier()

        @pl.when(sid < 10)
        def _():
            o = pl.multiple_of(sid * 1000, 8)
            for j, acc in enumerate((a00, a10, a01, a11)):
                pltpu.sync_copy(acc.at[pl.ds(o, 1000)],
                                out_h.at[cid, j, pl.ds(o, 1000)])

    return k(s0, d0, s1, d1, zdeg)


def _sc_messages(h0, h1, s0, d0, s1, d1, zagg):
    mesh = plsc.VectorSubcoreMesh(core_axis_name="c", subcore_axis_name="s")

    @functools.partial(
        pl.kernel,
        out_type=jax.ShapeDtypeStruct((NC, 2, N, F), jnp.float32),
        mesh=mesh,
        scratch_types=[
            pltpu.VMEM_SHARED((NP, F), jnp.float32),
            pltpu.VMEM_SHARED((NP, F), jnp.float32),
            pltpu.VMEM((ROWS_PER_TILE, LANE), jnp.int32),
            pltpu.VMEM((ROWS_PER_TILE, LANE), jnp.int32),
            pltpu.VMEM((LANE, F), jnp.float32),
            pltpu.SemaphoreType.DMA,
        ],
        compiler_params=pltpu.CompilerParams(use_tc_tiling_on_sc=False),
    )
    def k(h0_h, h1_h, s0_h, d0_h, s1_h, d1_h, zagg_h, out_h,
          agg0, agg1, idxs_v, idxd_v, rows_v, sem):
        cid = lax.axis_index("c")
        sid = lax.axis_index("s")

        @pl.when(sid == 0)
        def _():
            pltpu.sync_copy(zagg_h, agg0)
            pltpu.sync_copy(zagg_h, agg1)
        plsc.subcore_barrier()

        rowbase = (cid * NS + sid) * ROWS_PER_TILE
        for sh, dh, hh, agg in ((s0_h, d0_h, h0_h, agg0),
                                (s1_h, d1_h, h1_h, agg1)):
            pltpu.sync_copy(sh.at[pl.ds(rowbase, ROWS_PER_TILE)], idxs_v)
            pltpu.sync_copy(dh.at[pl.ds(rowbase, ROWS_PER_TILE)], idxd_v)

            def body(j, _, hh=hh, agg=agg):
                pltpu.async_copy(hh.at[idxs_v.at[j]], rows_v, sem).wait()
                pltpu.sync_copy(rows_v, agg.at[idxd_v.at[j]], add=True)
                return 0
            lax.fori_loop(0, ROWS_PER_TILE, body, 0)
        plsc.subcore_barrier()

        @pl.when(sid < 10)
        def _():
            o = pl.multiple_of(sid * 1000, 8)
            for g, agg in enumerate((agg0, agg1)):
                pltpu.sync_copy(agg.at[pl.ds(o, 1000)],
                                out_h.at[cid, g, pl.ds(o, 1000)])

    return k(h0, h1, s0, d0, s1, d1, zagg)


def _tc_h(x_st, od_st):
    def body(x_ref, od_ref, h_ref):
        deg = od_ref[0, :, 0:1] + od_ref[0, :, 1:2]
        norm = lax.rsqrt(jnp.maximum(deg, 1.0))
        h_ref[0] = x_ref[0] * norm

    return pl.pallas_call(
        body,
        grid=(2,),
        in_specs=[
            pl.BlockSpec((1, NP, F), lambda g: (g, 0, 0)),
            pl.BlockSpec((1, NP, 2), lambda g: (g, 0, 0)),
        ],
        out_specs=pl.BlockSpec((1, NP, F), lambda g: (g, 0, 0)),
        out_shape=jax.ShapeDtypeStruct((2, NP, F), jnp.float32),
    )(x_st, od_st)


def _tc_final(agg_parts, id_st, W, b2):
    def body(ap_ref, id_ref, w_ref, b_ref, out_ref):
        agg = ap_ref[0, 0] + ap_ref[1, 0]
        deg = id_ref[0, :, 0:1] + id_ref[0, :, 1:2]
        norm = lax.rsqrt(jnp.maximum(deg, 1.0))
        out_ref[0] = jnp.dot(agg * norm, w_ref[...],
                             preferred_element_type=jnp.float32) + b_ref[...]

    return pl.pallas_call(
        body,
        grid=(2,),
        in_specs=[
            pl.BlockSpec((NC, 1, N, F), lambda g: (0, g, 0, 0)),
            pl.BlockSpec((1, N, 2), lambda g: (g, 0, 0)),
            pl.BlockSpec((F, H), lambda g: (0, 0)),
            pl.BlockSpec((1, H), lambda g: (0, 0)),
        ],
        out_specs=pl.BlockSpec((1, N, H), lambda g: (g, 0, 0)),
        out_shape=jax.ShapeDtypeStruct((2, N, H), jnp.float32),
    )(agg_parts, id_st, W, b2)


def kernel(feats0, feats1, W, b, edge_index0, edge_index1):
    pad = jnp.full((EPAD - E,), N, jnp.int32)
    s0 = jnp.concatenate([edge_index0[0], pad]).reshape(EROWS, LANE)
    d0 = jnp.concatenate([edge_index0[1], pad]).reshape(EROWS, LANE)
    s1 = jnp.concatenate([edge_index1[0], pad]).reshape(EROWS, LANE)
    d1 = jnp.concatenate([edge_index1[1], pad]).reshape(EROWS, LANE)
    zdeg = jnp.zeros((NP,), jnp.float32)
    zagg = jnp.zeros((NP, F), jnp.float32)

    dp = _sc_degrees(s0, d0, s1, d1, zdeg)
    dpp = jnp.pad(dp, ((0, 0), (0, 0), (0, NP - N)))
    od_st = jnp.stack([dpp[:, 0, :].T, dpp[:, 2, :].T])
    id_st = jnp.stack([dp[:, 1, :].T, dp[:, 3, :].T])

    x_st = jnp.stack([jnp.pad(feats0, ((0, NP - N), (0, 0))),
                      jnp.pad(feats1, ((0, NP - N), (0, 0)))])
    h_st = _tc_h(x_st, od_st)

    ap = _sc_messages(h_st[0], h_st[1], s0, d0, s1, d1, zagg)
    out = _tc_final(ap, id_st, W, b.reshape(1, H))
    return (out[0], out[1])


# speedup vs baseline: 6.2490x; 1.1850x over previous
import functools

import jax
import jax.numpy as jnp
from jax import lax
from jax.experimental import pallas as pl
from jax.experimental.pallas import tpu as pltpu
from jax.experimental.pallas import tpu_sc as plsc

N = 10000
E = 320000
F = 64
H = 128
NC = 2
NS = 16

LANE = 128
EROWS = 2560
EPAD = EROWS * LANE
ROWS_PER_TILE = EROWS // (NC * NS)
NP = N + 8


def _sc_degrees(s0, d0, s1, d1, zdeg):
    mesh = plsc.VectorSubcoreMesh(core_axis_name="c", subcore_axis_name="s")

    @functools.partial(
        pl.kernel,
        out_type=jax.ShapeDtypeStruct((NC, 4, N), jnp.float32),
        mesh=mesh,
        scratch_types=[
            pltpu.VMEM_SHARED((NP,), jnp.float32),
            pltpu.VMEM_SHARED((NP,), jnp.float32),
            pltpu.VMEM_SHARED((NP,), jnp.float32),
            pltpu.VMEM_SHARED((NP,), jnp.float32),
            pltpu.VMEM((ROWS_PER_TILE, LANE), jnp.int32),
            pltpu.VMEM((LANE,), jnp.float32),
        ],
        compiler_params=pltpu.CompilerParams(use_tc_tiling_on_sc=False),
    )
    def k(s0_h, d0_h, s1_h, d1_h, zdeg_h, out_h,
          a00, a10, a01, a11, idx_v, ones_v):
        cid = lax.axis_index("c")
        sid = lax.axis_index("s")

        @pl.when(sid == 0)
        def _():
            pltpu.sync_copy(zdeg_h, a00)
            pltpu.sync_copy(zdeg_h, a10)
            pltpu.sync_copy(zdeg_h, a01)
            pltpu.sync_copy(zdeg_h, a11)

        for i in range(LANE // 16):
            ones_v[pl.ds(i * 16, 16)] = jnp.full((16,), 1.0, jnp.float32)
        plsc.subcore_barrier()

        rowbase = (cid * NS + sid) * ROWS_PER_TILE
        for eh, acc in ((s0_h, a00), (d0_h, a10), (s1_h, a01), (d1_h, a11)):
            pltpu.sync_copy(eh.at[pl.ds(rowbase, ROWS_PER_TILE)], idx_v)

            def body(j, _, acc=acc):
                pltpu.sync_copy(ones_v, acc.at[idx_v.at[j]], add=True)
                return 0
            lax.fori_loop(0, ROWS_PER_TILE, body, 0)
        plsc.subcore_barrier()

        @pl.when(sid < 10)
        def _():
            o = pl.multiple_of(sid * 1000, 8)
            for j, acc in enumerate((a00, a10, a01, a11)):
                pltpu.sync_copy(acc.at[pl.ds(o, 1000)],
                                out_h.at[cid, j, pl.ds(o, 1000)])

    return k(s0, d0, s1, d1, zdeg)


def _sc_messages(h0, h1, s0, d0, s1, d1, zagg):
    mesh = plsc.VectorSubcoreMesh(core_axis_name="c", subcore_axis_name="s")

    @functools.partial(
        pl.kernel,
        out_type=jax.ShapeDtypeStruct((NC, 2, N, F), jnp.float32),
        mesh=mesh,
        scratch_types=[
            pltpu.VMEM_SHARED((NP, F), jnp.float32),
            pltpu.VMEM_SHARED((NP, F), jnp.float32),
            pltpu.VMEM((ROWS_PER_TILE, LANE), jnp.int32),
            pltpu.VMEM((ROWS_PER_TILE, LANE), jnp.int32),
            pltpu.VMEM((LANE, F), jnp.float32),
            pltpu.VMEM((LANE, F), jnp.float32),
            pltpu.SemaphoreType.DMA,
            pltpu.SemaphoreType.DMA,
        ],
        compiler_params=pltpu.CompilerParams(use_tc_tiling_on_sc=False),
    )
    def k(h0_h, h1_h, s0_h, d0_h, s1_h, d1_h, zagg_h, out_h,
          agg0, agg1, idxs_v, idxd_v, rows_a, rows_b, sem_a, sem_b):
        cid = lax.axis_index("c")
        sid = lax.axis_index("s")

        @pl.when(sid == 0)
        def _():
            pltpu.sync_copy(zagg_h, agg0)
            pltpu.sync_copy(zagg_h, agg1)
        plsc.subcore_barrier()

        rowbase = (cid * NS + sid) * ROWS_PER_TILE
        for sh, dh, hh, agg in ((s0_h, d0_h, h0_h, agg0),
                                (s1_h, d1_h, h1_h, agg1)):
            pltpu.sync_copy(sh.at[pl.ds(rowbase, ROWS_PER_TILE)], idxs_v)
            pltpu.sync_copy(dh.at[pl.ds(rowbase, ROWS_PER_TILE)], idxd_v)

            def gstart(r, buf, sem, hh=hh):
                pltpu.async_copy(hh.at[idxs_v.at[r]], buf, sem)

            def gwait(r, buf, sem, hh=hh):
                pltpu.make_async_copy(hh.at[idxs_v.at[r]], buf, sem).wait()

            gstart(0, rows_a, sem_a)

            def body(j2, _, hh=hh, agg=agg, gstart=gstart, gwait=gwait):
                r0 = j2 * 2
                r1 = r0 + 1
                gstart(r1, rows_b, sem_b)
                gwait(r0, rows_a, sem_a)
                pltpu.sync_copy(rows_a, agg.at[idxd_v.at[r0]], add=True)

                @pl.when(j2 < ROWS_PER_TILE // 2 - 1)
                def _():
                    gstart(r0 + 2, rows_a, sem_a)
                gwait(r1, rows_b, sem_b)
                pltpu.sync_copy(rows_b, agg.at[idxd_v.at[r1]], add=True)
                return 0
            lax.fori_loop(0, ROWS_PER_TILE // 2, body, 0)
        plsc.subcore_barrier()

        @pl.when(sid < 10)
        def _():
            o = pl.multiple_of(sid * 1000, 8)
            for g, agg in enumerate((agg0, agg1)):
                pltpu.sync_copy(agg.at[pl.ds(o, 1000)],
                                out_h.at[cid, g, pl.ds(o, 1000)])

    return k(h0, h1, s0, d0, s1, d1, zagg)


def _tc_h(x_st, od_st):
    def body(x_ref, od_ref, h_ref):
        deg = od_ref[0, :, 0:1] + od_ref[0, :, 1:2]
        norm = lax.rsqrt(jnp.maximum(deg, 1.0))
        h_ref[0] = x_ref[0] * norm

    return pl.pallas_call(
        body,
        grid=(2,),
        in_specs=[
            pl.BlockSpec((1, NP, F), lambda g: (g, 0, 0)),
            pl.BlockSpec((1, NP, 2), lambda g: (g, 0, 0)),
        ],
        out_specs=pl.BlockSpec((1, NP, F), lambda g: (g, 0, 0)),
        out_shape=jax.ShapeDtypeStruct((2, NP, F), jnp.float32),
    )(x_st, od_st)


def _tc_final(agg_parts, id_st, W, b2):
    def body(ap_ref, id_ref, w_ref, b_ref, out_ref):
        agg = ap_ref[0, 0] + ap_ref[1, 0]
        deg = id_ref[0, :, 0:1] + id_ref[0, :, 1:2]
        norm = lax.rsqrt(jnp.maximum(deg, 1.0))
        out_ref[0] = jnp.dot(agg * norm, w_ref[...],
                             preferred_element_type=jnp.float32) + b_ref[...]

    return pl.pallas_call(
        body,
        grid=(2,),
        in_specs=[
            pl.BlockSpec((NC, 1, N, F), lambda g: (0, g, 0, 0)),
            pl.BlockSpec((1, N, 2), lambda g: (g, 0, 0)),
            pl.BlockSpec((F, H), lambda g: (0, 0)),
            pl.BlockSpec((1, H), lambda g: (0, 0)),
        ],
        out_specs=pl.BlockSpec((1, N, H), lambda g: (g, 0, 0)),
        out_shape=jax.ShapeDtypeStruct((2, N, H), jnp.float32),
    )(agg_parts, id_st, W, b2)


def kernel(feats0, feats1, W, b, edge_index0, edge_index1):
    pad = jnp.full((EPAD - E,), N, jnp.int32)
    s0 = jnp.concatenate([edge_index0[0], pad]).reshape(EROWS, LANE)
    d0 = jnp.concatenate([edge_index0[1], pad]).reshape(EROWS, LANE)
    s1 = jnp.concatenate([edge_index1[0], pad]).reshape(EROWS, LANE)
    d1 = jnp.concatenate([edge_index1[1], pad]).reshape(EROWS, LANE)
    zdeg = jnp.zeros((NP,), jnp.float32)
    zagg = jnp.zeros((NP, F), jnp.float32)

    dp = _sc_degrees(s0, d0, s1, d1, zdeg)
    dpp = jnp.pad(dp, ((0, 0), (0, 0), (0, NP - N)))
    od_st = jnp.stack([dpp[:, 0, :].T, dpp[:, 2, :].T])
    id_st = jnp.stack([dp[:, 1, :].T, dp[:, 3, :].T])

    x_st = jnp.stack([jnp.pad(feats0, ((0, NP - N), (0, 0))),
                      jnp.pad(feats1, ((0, NP - N), (0, 0)))])
    h_st = _tc_h(x_st, od_st)

    ap = _sc_messages(h_st[0], h_st[1], s0, d0, s1, d1, zagg)
    out = _tc_final(ap, id_st, W, b.reshape(1, H))
    return (out[0], out[1])
